# baseline (speedup 1.0000x reference)
import functools

import jax
import jax.numpy as jnp
from jax import lax
from jax.experimental import pallas as pl
from jax.experimental.pallas import tpu as pltpu
from jax.experimental.pallas import tpu_sc as plsc

B, L = 4096, 200
VOCAB_SIZE, CONCEPT_DIM = 32, 8
N_EMB, D_EMB = 64, 16

_INFO = plsc.get_sparse_core_info()
_NC, _NS, _LANES = _INFO.num_cores, _INFO.num_subcores, _INFO.num_lanes
_NW = _NC * _NS
_RW = B // _NW
_GROUPS = _RW // _LANES


def _sc_hist(ids_hbm, out_hbm, ids_v, counts_v):
    wid = lax.axis_index("s") * _NC + lax.axis_index("c")
    base = wid * _RW
    pltpu.sync_copy(ids_hbm.at[pl.ds(base, _RW)], ids_v)

    zeros = jnp.zeros((_LANES,), jnp.float32)

    def _zero(r, carry):
        for j in range(N_EMB // _LANES):
            counts_v[r, pl.ds(j * _LANES, _LANES)] = zeros
        return carry
    lax.fori_loop(0, _RW, _zero, 0)

    ones = jnp.ones((_LANES,), jnp.float32)
    lane_iota = lax.iota(jnp.int32, _LANES)
    for g in range(_GROUPS):
        row_idx = g * _LANES + lane_iota
        def _step(l, carry, row_idx=row_idx):
            l_splat = jnp.full((_LANES,), l, jnp.int32)
            ids16 = plsc.load_gather(ids_v, [row_idx, l_splat])
            plsc.addupdate_scatter(counts_v, [row_idx, ids16], ones)
            return carry
        lax.fori_loop(0, L, _step, 0)

    pltpu.sync_copy(counts_v, out_hbm.at[pl.ds(base, _RW)])


@functools.partial(
    pl.kernel,
    out_type=jax.ShapeDtypeStruct((B, N_EMB), jnp.float32),
    scratch_types=[
        pltpu.VMEM((_RW, L), jnp.int32),
        pltpu.VMEM((_RW, N_EMB), jnp.float32),
    ],
    mesh=plsc.VectorSubcoreMesh(core_axis_name="c", subcore_axis_name="s"),
    compiler_params=pltpu.CompilerParams(needs_layout_passes=False),
)
def _sc_counts(ids_hbm, out_hbm, ids_v, counts_v):
    _sc_hist(ids_hbm, out_hbm, ids_v, counts_v)


TC_BLOCK = 1024


def _tc_heads(counts_ref, embed_ref, wh_ref, bh_ref, wc_ref, bc_ref,
              logits_ref, conc_ref):
    counts = counts_ref[...]
    x = jnp.dot(counts, embed_ref[...], preferred_element_type=jnp.float32)
    x = x * (1.0 / L)
    logits_ref[...] = jnp.dot(
        x, wh_ref[...], preferred_element_type=jnp.float32) + bh_ref[...]
    conc_ref[...] = jnp.dot(
        x, wc_ref[...], preferred_element_type=jnp.float32) + bc_ref[...]


@jax.jit
def kernel(input_ids, embed, W_head, b_head, W_concept, b_concept):
    counts = _sc_counts(input_ids)
    logits2d, concepts = pl.pallas_call(
        _tc_heads,
        grid=(B // TC_BLOCK,),
        in_specs=[
            pl.BlockSpec((TC_BLOCK, N_EMB), lambda i: (i, 0)),
            pl.BlockSpec((N_EMB, D_EMB), lambda i: (0, 0)),
            pl.BlockSpec((D_EMB, VOCAB_SIZE), lambda i: (0, 0)),
            pl.BlockSpec((1, VOCAB_SIZE), lambda i: (0, 0)),
            pl.BlockSpec((D_EMB, CONCEPT_DIM), lambda i: (0, 0)),
            pl.BlockSpec((1, CONCEPT_DIM), lambda i: (0, 0)),
        ],
        out_specs=[
            pl.BlockSpec((TC_BLOCK, VOCAB_SIZE), lambda i: (i, 0)),
            pl.BlockSpec((TC_BLOCK, CONCEPT_DIM), lambda i: (i, 0)),
        ],
        out_shape=[
            jax.ShapeDtypeStruct((B, VOCAB_SIZE), jnp.float32),
            jax.ShapeDtypeStruct((B, CONCEPT_DIM), jnp.float32),
        ],
    )(counts, embed, W_head, b_head.reshape(1, VOCAB_SIZE),
      W_concept, b_concept.reshape(1, CONCEPT_DIM))
    logits = jnp.broadcast_to(logits2d[:, None, :], (B, L, VOCAB_SIZE))
    vertex_preds = jnp.zeros((B, L), dtype=jnp.int32)
    return (logits, concepts, vertex_preds)


# speedup vs baseline: 1.0166x; 1.0166x over previous
import functools

import jax
import jax.numpy as jnp
from jax import lax
from jax.experimental import pallas as pl
from jax.experimental.pallas import tpu as pltpu
from jax.experimental.pallas import tpu_sc as plsc

B, L = 4096, 200
VOCAB_SIZE, CONCEPT_DIM = 32, 8
N_EMB, D_EMB = 64, 16

_INFO = plsc.get_sparse_core_info()
_NC, _NS, _LANES = _INFO.num_cores, _INFO.num_subcores, _INFO.num_lanes
_NW = _NC * _NS
_RW = B // _NW
_GROUPS = _RW // _LANES


def _sc_hist(ids_hbm, out_hbm, ids_v, counts_v):
    wid = lax.axis_index("s") * _NC + lax.axis_index("c")
    base = wid * _RW
    pltpu.sync_copy(ids_hbm.at[pl.ds(base * L, _RW * L)], ids_v)

    zeros = jnp.zeros((_LANES,), jnp.float32)

    def _zero(i, carry):
        counts_v[pl.ds(i * _LANES, _LANES)] = zeros
        return carry
    lax.fori_loop(0, _RW * N_EMB // _LANES, _zero, 0)

    ones = jnp.ones((_LANES,), jnp.float32)
    lane_iota = lax.iota(jnp.int32, _LANES)
    for g in range(_GROUPS):
        row_idx = g * _LANES + lane_iota
        id_off = row_idx * L
        cnt_off = row_idx * N_EMB

        def _step(l, carry, id_off=id_off, cnt_off=cnt_off):
            ids16 = plsc.load_gather(ids_v, [id_off + l])
            plsc.addupdate_scatter(counts_v, [cnt_off + ids16], ones)
            return carry
        lax.fori_loop(0, L, _step, 0)

    pltpu.sync_copy(counts_v, out_hbm.at[pl.ds(base * N_EMB, _RW * N_EMB)])


@functools.partial(
    pl.kernel,
    out_type=jax.ShapeDtypeStruct((B * N_EMB,), jnp.float32),
    scratch_types=[
        pltpu.VMEM((_RW * L,), jnp.int32),
        pltpu.VMEM((_RW * N_EMB,), jnp.float32),
    ],
    mesh=plsc.VectorSubcoreMesh(core_axis_name="c", subcore_axis_name="s"),
    compiler_params=pltpu.CompilerParams(needs_layout_passes=False),
)
def _sc_counts(ids_hbm, out_hbm, ids_v, counts_v):
    _sc_hist(ids_hbm, out_hbm, ids_v, counts_v)


TC_BLOCK = 1024


def _tc_heads(counts_ref, embed_ref, wh_ref, bh_ref, wc_ref, bc_ref,
              logits_ref, conc_ref):
    counts = counts_ref[...]
    x = jnp.dot(counts, embed_ref[...], preferred_element_type=jnp.float32)
    x = x * (1.0 / L)
    logits_ref[...] = jnp.dot(
        x, wh_ref[...], preferred_element_type=jnp.float32) + bh_ref[...]
    conc_ref[...] = jnp.dot(
        x, wc_ref[...], preferred_element_type=jnp.float32) + bc_ref[...]


@jax.jit
def kernel(input_ids, embed, W_head, b_head, W_concept, b_concept):
    counts_flat = _sc_counts(input_ids.reshape(B * L))
    counts = counts_flat.reshape(B, N_EMB)
    logits2d, concepts = pl.pallas_call(
        _tc_heads,
        grid=(B // TC_BLOCK,),
        in_specs=[
            pl.BlockSpec((TC_BLOCK, N_EMB), lambda i: (i, 0)),
            pl.BlockSpec((N_EMB, D_EMB), lambda i: (0, 0)),
            pl.BlockSpec((D_EMB, VOCAB_SIZE), lambda i: (0, 0)),
            pl.BlockSpec((1, VOCAB_SIZE), lambda i: (0, 0)),
            pl.BlockSpec((D_EMB, CONCEPT_DIM), lambda i: (0, 0)),
            pl.BlockSpec((1, CONCEPT_DIM), lambda i: (0, 0)),
        ],
        out_specs=[
            pl.BlockSpec((TC_BLOCK, VOCAB_SIZE), lambda i: (i, 0)),
            pl.BlockSpec((TC_BLOCK, CONCEPT_DIM), lambda i: (i, 0)),
        ],
        out_shape=[
            jax.ShapeDtypeStruct((B, VOCAB_SIZE), jnp.float32),
            jax.ShapeDtypeStruct((B, CONCEPT_DIM), jnp.float32),
        ],
    )(counts, embed, W_head, b_head.reshape(1, VOCAB_SIZE),
      W_concept, b_concept.reshape(1, CONCEPT_DIM))
    logits = jnp.broadcast_to(logits2d[:, None, :], (B, L, VOCAB_SIZE))
    vertex_preds = jnp.zeros((B, L), dtype=jnp.int32)
    return (logits, concepts, vertex_preds)
